# Initial kernel scaffold; baseline (speedup 1.0000x reference)
import jax
import jax.numpy as jnp
from jax.experimental import pallas as pl


def kernel(z, W):
    raise NotImplementedError("write your pallas kernel here")


# speedup vs baseline: 1.3526x; 1.3526x over previous
import functools

import jax
import jax.numpy as jnp
from jax.experimental import pallas as pl
from jax.experimental.pallas import tpu as pltpu

_K = 8192
_D = 128
_BETA = 0.25
_TEMP = 0.1
_KT = 512
_KB = _K // _KT


def _vq_body(z_ref, w_ref, zq_ref, com_ref, ppl_ref, ent_ref,
             zn_ref, m_ref, s_ref, idx_ref, n_rows):
    p = pl.program_id(0)
    k = pl.program_id(1)

    @pl.when(jnp.logical_and(p == 0, k == 0))
    def _init():
        z = z_ref[...]
        nrm = jnp.sqrt(jnp.sum(z * z, axis=1, keepdims=True))
        zn_ref[...] = z / jnp.maximum(nrm, 1e-12)

    w = w_ref[...]
    wn_nrm = jnp.sqrt(jnp.sum(w * w, axis=1, keepdims=True))
    wn = w / jnp.maximum(wn_nrm, 1e-12)
    l = jax.lax.dot_general(
        zn_ref[...], wn, (((1,), (1,)), ((), ())),
        preferred_element_type=jnp.float32) / _TEMP

    @pl.when(p == 0)
    def _pass_a():
        tmax = jnp.max(l, axis=1, keepdims=True)
        colidx = jax.lax.broadcasted_iota(jnp.int32, l.shape, 1)
        targ = jnp.min(jnp.where(l == tmax, colidx, _KT),
                       axis=1, keepdims=True) + k * _KT

        @pl.when(k == 0)
        def _first():
            m_ref[...] = tmax
            s_ref[...] = jnp.sum(jnp.exp(l - tmax), axis=1, keepdims=True)
            idx_ref[...] = targ

        @pl.when(k > 0)
        def _rest():
            m_old = m_ref[...]
            m_new = jnp.maximum(m_old, tmax)
            s_ref[...] = (s_ref[...] * jnp.exp(m_old - m_new)
                          + jnp.sum(jnp.exp(l - m_new), axis=1, keepdims=True))
            idx_ref[...] = jnp.where(tmax > m_old, targ, idx_ref[...])
            m_ref[...] = m_new

    @pl.when(p == 1)
    def _pass_b():
        probs = jnp.exp(l - m_ref[...]) / s_ref[...]
        pavg = jnp.sum(probs, axis=0, keepdims=True) / n_rows + 1e-8
        cols = jax.lax.broadcasted_iota(jnp.int32, l.shape, 1) + k * _KT
        oh = (idx_ref[...] == cols).astype(jnp.float32)
        e_mean = jnp.sum(oh, axis=0, keepdims=True) / n_rows
        zq_part = jax.lax.dot_general(
            oh, w, (((1,), (0,)), ((), ())),
            preferred_element_type=jnp.float32)

        ent_part = jnp.sum(pavg * jnp.log(pavg)).reshape(1, 1)
        ppl_part = jnp.sum(e_mean * jnp.log(e_mean + 1e-8)).reshape(1, 1)

        @pl.when(k == 0)
        def _first():
            zq_ref[...] = zq_part
            ent_ref[...] = ent_part
            ppl_ref[...] = ppl_part

        @pl.when(k > 0)
        def _rest():
            zq_ref[...] += zq_part
            ent_ref[...] += ent_part
            ppl_ref[...] += ppl_part

        @pl.when(k == _KB - 1)
        def _finalize():
            diff = zq_ref[...] - z_ref[...]
            com_ref[...] = ((1.0 + _BETA)
                            * jnp.sum(diff * diff) / (n_rows * _D)
                            ).reshape(1, 1)
            ent_ref[...] = -ent_ref[...]
            ppl_ref[...] = jnp.exp(-ppl_ref[...])


@functools.partial(jax.jit, static_argnames=("interpret",))
def _cos_vq(z_flat, W, interpret=False):
    n = z_flat.shape[0]
    zq, com, ppl, ent = pl.pallas_call(
        functools.partial(_vq_body, n_rows=n),
        grid=(2, _KB),
        in_specs=[
            pl.BlockSpec((n, _D), lambda p, k: (0, 0)),
            pl.BlockSpec((_KT, _D), lambda p, k: (k, 0)),
        ],
        out_specs=[
            pl.BlockSpec((n, _D), lambda p, k: (0, 0)),
            pl.BlockSpec((1, 1), lambda p, k: (0, 0)),
            pl.BlockSpec((1, 1), lambda p, k: (0, 0)),
            pl.BlockSpec((1, 1), lambda p, k: (0, 0)),
        ],
        out_shape=[
            jax.ShapeDtypeStruct((n, _D), jnp.float32),
            jax.ShapeDtypeStruct((1, 1), jnp.float32),
            jax.ShapeDtypeStruct((1, 1), jnp.float32),
            jax.ShapeDtypeStruct((1, 1), jnp.float32),
        ],
        scratch_shapes=[
            pltpu.VMEM((n, _D), jnp.float32),
            pltpu.VMEM((n, 1), jnp.float32),
            pltpu.VMEM((n, 1), jnp.float32),
            pltpu.VMEM((n, 1), jnp.int32),
        ],
        interpret=interpret,
    )(z_flat, W)
    return zq, com[0, 0], ppl[0, 0], ent[0, 0]


def kernel(z, W):
    z_flat = z.reshape(-1, _D)
    zq, com, ppl, ent = _cos_vq(z_flat, W)
    return zq.reshape(z.shape), com, ppl, ent


# speedup vs baseline: 1.8551x; 1.3716x over previous
import functools

import jax
import jax.numpy as jnp
from jax.experimental import pallas as pl
from jax.experimental.pallas import tpu as pltpu

_K = 8192
_D = 128
_BETA = 0.25
_TEMP = 0.1
_NB = 512


def _vq_body(z_ref, w_ref, zq_ref, com_ref, ppl_ref, ent_ref,
             wn_ref, psum_ref, cnt_ref, com_acc, n_rows, rb):
    r = pl.program_id(0)

    @pl.when(r == 0)
    def _init():
        w = w_ref[...]
        nrm = jnp.sqrt(jnp.sum(w * w, axis=1, keepdims=True))
        wn_ref[...] = w / jnp.maximum(nrm, 1e-12)
        psum_ref[...] = jnp.zeros_like(psum_ref)
        cnt_ref[...] = jnp.zeros_like(cnt_ref)
        com_acc[...] = jnp.zeros_like(com_acc)

    z = z_ref[...]
    znrm = jnp.sqrt(jnp.sum(z * z, axis=1, keepdims=True))
    zn = z / jnp.maximum(znrm, 1e-12)
    c = jax.lax.dot_general(zn, wn_ref[...], (((1,), (1,)), ((), ())),
                            preferred_element_type=jnp.float32)
    m = jnp.max(c, axis=1, keepdims=True)
    colidx = jax.lax.broadcasted_iota(jnp.int32, c.shape, 1)
    idx = jnp.min(jnp.where(c == m, colidx, _K), axis=1, keepdims=True)
    e = jnp.exp((c - m) / _TEMP)
    s = jnp.sum(e, axis=1, keepdims=True)
    psum_ref[...] += jnp.sum(e * (1.0 / s), axis=0, keepdims=True)
    oh = (colidx == idx).astype(jnp.float32)
    cnt_ref[...] += jnp.sum(oh, axis=0, keepdims=True)
    zq = jax.lax.dot_general(oh, w_ref[...], (((1,), (0,)), ((), ())),
                             preferred_element_type=jnp.float32)
    zq_ref[...] = zq
    diff = zq - z
    com_acc[...] += jnp.sum(diff * diff).reshape(1, 1)

    @pl.when(r == rb - 1)
    def _finalize():
        pavg = psum_ref[...] / n_rows + 1e-8
        ent_ref[...] = -jnp.sum(pavg * jnp.log(pavg)).reshape(1, 1)
        e_mean = cnt_ref[...] / n_rows
        ppl_ref[...] = jnp.exp(
            -jnp.sum(e_mean * jnp.log(e_mean + 1e-8))).reshape(1, 1)
        com_ref[...] = (1.0 + _BETA) * com_acc[...] / (n_rows * _D)


@jax.jit
def _cos_vq(z_flat, W):
    n = z_flat.shape[0]
    rb = n // _NB
    zq, com, ppl, ent = pl.pallas_call(
        functools.partial(_vq_body, n_rows=n, rb=rb),
        grid=(rb,),
        in_specs=[
            pl.BlockSpec((_NB, _D), lambda r: (r, 0)),
            pl.BlockSpec((_K, _D), lambda r: (0, 0)),
        ],
        out_specs=[
            pl.BlockSpec((_NB, _D), lambda r: (r, 0)),
            pl.BlockSpec((1, 1), lambda r: (0, 0)),
            pl.BlockSpec((1, 1), lambda r: (0, 0)),
            pl.BlockSpec((1, 1), lambda r: (0, 0)),
        ],
        out_shape=[
            jax.ShapeDtypeStruct((n, _D), jnp.float32),
            jax.ShapeDtypeStruct((1, 1), jnp.float32),
            jax.ShapeDtypeStruct((1, 1), jnp.float32),
            jax.ShapeDtypeStruct((1, 1), jnp.float32),
        ],
        scratch_shapes=[
            pltpu.VMEM((_K, _D), jnp.float32),
            pltpu.VMEM((1, _K), jnp.float32),
            pltpu.VMEM((1, _K), jnp.float32),
            pltpu.VMEM((1, 1), jnp.float32),
        ],
    )(z_flat, W)
    return zq, com[0, 0], ppl[0, 0], ent[0, 0]


def kernel(z, W):
    z_flat = z.reshape(-1, _D)
    zq, com, ppl, ent = _cos_vq(z_flat, W)
    return zq.reshape(z.shape), com, ppl, ent


# speedup vs baseline: 3.0157x; 1.6256x over previous
import functools

import jax
import jax.numpy as jnp
from jax.experimental import pallas as pl
from jax.experimental.pallas import tpu as pltpu

_K = 8192
_D = 128
_BETA = 0.25
_TEMP = 0.1
_NB = 512


def _vq_body(z_ref, w_ref, zq_ref, com_ref, ppl_ref, ent_ref,
             wn_ref, psum_ref, cnt_ref, com_acc, n_rows, rb):
    r = pl.program_id(0)

    @pl.when(r == 0)
    def _init():
        w = w_ref[...]
        nrm = jnp.sqrt(jnp.sum(w * w, axis=1, keepdims=True))
        wn_ref[...] = w / jnp.maximum(nrm, 1e-12)
        psum_ref[...] = jnp.zeros_like(psum_ref)
        cnt_ref[...] = jnp.zeros_like(cnt_ref)
        com_acc[...] = jnp.zeros_like(com_acc)

    z = z_ref[...]
    znrm = jnp.sqrt(jnp.sum(z * z, axis=1, keepdims=True))
    zn = z / jnp.maximum(znrm, 1e-12)
    c = jax.lax.dot_general(zn, wn_ref[...], (((1,), (1,)), ((), ())),
                            preferred_element_type=jnp.float32)
    idx = jnp.argmax(c, axis=1).astype(jnp.int32).reshape(-1, 1)
    e = jnp.exp(c * (1.0 / _TEMP))
    ones_k = jnp.ones((_K, 1), jnp.float32)
    s = jax.lax.dot_general(e, ones_k, (((1,), (0,)), ((), ())),
                            preferred_element_type=jnp.float32)
    psum_ref[...] += jax.lax.dot_general(
        1.0 / s, e, (((0,), (0,)), ((), ())),
        preferred_element_type=jnp.float32)
    colidx = jax.lax.broadcasted_iota(jnp.int32, c.shape, 1)
    oh = (colidx == idx).astype(jnp.float32)
    ones_n = jnp.ones((oh.shape[0], 1), jnp.float32)
    cnt_ref[...] += jax.lax.dot_general(
        ones_n, oh, (((0,), (0,)), ((), ())),
        preferred_element_type=jnp.float32)
    zq = jax.lax.dot_general(oh, w_ref[...], (((1,), (0,)), ((), ())),
                             preferred_element_type=jnp.float32)
    zq_ref[...] = zq
    diff = zq - z
    com_acc[...] += jnp.sum(diff * diff).reshape(1, 1)

    @pl.when(r == rb - 1)
    def _finalize():
        pavg = psum_ref[...] / n_rows + 1e-8
        ent_ref[...] = -jnp.sum(pavg * jnp.log(pavg)).reshape(1, 1)
        e_mean = cnt_ref[...] / n_rows
        ppl_ref[...] = jnp.exp(
            -jnp.sum(e_mean * jnp.log(e_mean + 1e-8))).reshape(1, 1)
        com_ref[...] = (1.0 + _BETA) * com_acc[...] / (n_rows * _D)


@jax.jit
def _cos_vq(z_flat, W):
    n = z_flat.shape[0]
    rb = n // _NB
    zq, com, ppl, ent = pl.pallas_call(
        functools.partial(_vq_body, n_rows=n, rb=rb),
        grid=(rb,),
        in_specs=[
            pl.BlockSpec((_NB, _D), lambda r: (r, 0)),
            pl.BlockSpec((_K, _D), lambda r: (0, 0)),
        ],
        out_specs=[
            pl.BlockSpec((_NB, _D), lambda r: (r, 0)),
            pl.BlockSpec((1, 1), lambda r: (0, 0)),
            pl.BlockSpec((1, 1), lambda r: (0, 0)),
            pl.BlockSpec((1, 1), lambda r: (0, 0)),
        ],
        out_shape=[
            jax.ShapeDtypeStruct((n, _D), jnp.float32),
            jax.ShapeDtypeStruct((1, 1), jnp.float32),
            jax.ShapeDtypeStruct((1, 1), jnp.float32),
            jax.ShapeDtypeStruct((1, 1), jnp.float32),
        ],
        scratch_shapes=[
            pltpu.VMEM((_K, _D), jnp.float32),
            pltpu.VMEM((1, _K), jnp.float32),
            pltpu.VMEM((1, _K), jnp.float32),
            pltpu.VMEM((1, 1), jnp.float32),
        ],
    )(z_flat, W)
    return zq, com[0, 0], ppl[0, 0], ent[0, 0]


def kernel(z, W):
    z_flat = z.reshape(-1, _D)
    zq, com, ppl, ent = _cos_vq(z_flat, W)
    return zq.reshape(z.shape), com, ppl, ent
